# baseline (speedup 1.0000x reference)
import functools

import jax
import jax.numpy as jnp
from jax import lax
from jax.experimental import pallas as pl
from jax.experimental.pallas import tpu as pltpu
from jax.experimental.pallas import tpu_sc as plsc

BM = 512
T = 64
NBUF = 2


def _mm_tanh_kernel(x_ref, w_ref, b_ref, o_ref):
    acc = lax.dot_general(x_ref[...], w_ref[...], (((1,), (1,)), ((), ())),
                          preferred_element_type=jnp.float32)
    h = jnp.tanh(acc + b_ref[...]).astype(jnp.bfloat16)
    d = h.shape[1]
    lo = lax.bitcast_convert_type(h[:, :d // 2], jnp.uint16)
    hi = lax.bitcast_convert_type(h[:, d // 2:], jnp.uint16)
    w = (hi.astype(jnp.uint32) << 16) | lo.astype(jnp.uint32)
    o_ref[...] = lax.bitcast_convert_type(w, jnp.int32)


def _compute_h(x, w1, b1):
    n, d = x.shape
    return pl.pallas_call(
        _mm_tanh_kernel,
        grid=(n // BM,),
        in_specs=[
            pl.BlockSpec((BM, d), lambda i: (i, 0)),
            pl.BlockSpec((d, d), lambda i: (0, 0)),
            pl.BlockSpec((1, d), lambda i: (0, 0)),
        ],
        out_specs=pl.BlockSpec((BM, d // 2), lambda i: (i, 0)),
        out_shape=jax.ShapeDtypeStruct((n, d // 2), jnp.int32),
    )(x, w1, b1)


@functools.partial(jax.jit, static_argnums=(3, 4, 5, 6))
def _sc_pool_dot(h, gidx3, w2p, ntok, d, f_, nc):
    ns = 16
    nw = nc * ns
    tpw = ntok // nw
    nblk = tpw // T
    dw = d // 2
    nch = dw // 16
    seq = gidx3.shape[-1]

    mesh = plsc.VectorSubcoreMesh(core_axis_name="c", subcore_axis_name="s")

    @functools.partial(
        pl.kernel,
        out_type=jax.ShapeDtypeStruct((ntok,), jnp.float32),
        mesh=mesh,
        scratch_types=[
            pltpu.VMEM((NBUF, f_, T, dw), jnp.int32),
            pltpu.VMEM((f_, tpw), jnp.int32),
            pltpu.VMEM((d + 16,), jnp.float32),
            pltpu.VMEM((tpw,), jnp.float32),
            pltpu.SemaphoreType.DMA,
            pltpu.SemaphoreType.DMA,
        ],
        compiler_params=pltpu.CompilerParams(use_tc_tiling_on_sc=False,
                                             needs_layout_passes=False),
    )
    def sc_fn(h_hbm, gidx_hbm, w2p_hbm, out_hbm, rows_v, idx_v, w2_v, out_v,
              sem0, sem1):
        wid = lax.axis_index("s") * nc + lax.axis_index("c")
        base_tok = wid * tpw
        bat = base_tok // seq
        s0 = base_tok - bat * seq
        sems = [sem0, sem1]

        pltpu.sync_copy(w2p_hbm, w2_v)
        b2 = w2_v[pl.ds(d, 16)][0]
        w2es = [w2_v[pl.ds(16 * c, 16)] for c in range(nch)]
        w2os = [w2_v[pl.ds(dw + 16 * c, 16)] for c in range(nch)]

        for ff in range(f_):
            pltpu.sync_copy(gidx_hbm.at[bat, ff, pl.ds(s0, tpw)],
                            idx_v.at[ff])
        boff = jnp.zeros((16,), jnp.int32) + bat * seq

        def off_body(i, carry):
            for ff in range(f_):
                sl = pl.ds(i * 16, 16)
                idx_v[ff, sl] = idx_v[ff, sl] + boff
            return carry

        lax.fori_loop(0, tpw // 16, off_body, 0)

        def issue(blk, s):
            for ff in range(f_):
                pltpu.async_copy(
                    h_hbm.at[idx_v.at[ff, pl.ds(blk * T, T)]],
                    rows_v.at[s, ff], sems[s])

        def wait(blk, s):
            for ff in range(f_):
                pltpu.make_async_copy(
                    h_hbm.at[idx_v.at[ff, pl.ds(blk * T, T)]],
                    rows_v.at[s, ff], sems[s]).wait()

        issue(0, 0)
        issue(1, 1)

        lane_iota = lax.iota(jnp.int32, 16)

        def compute_block(blk, s):
            rows = rows_v.at[s]

            def group_body(g, carry):
                def token_body(tl, resvec):
                    t = g * 16 + tl
                    acc0 = jnp.zeros((16,), jnp.float32)
                    acc1 = jnp.zeros((16,), jnp.float32)
                    for c in range(nch):
                        sl = pl.ds(16 * c, 16)
                        v0 = plsc.bitcast(rows[0, t, sl], jnp.bfloat16)
                        v1 = plsc.bitcast(rows[1, t, sl], jnp.bfloat16)
                        v2 = plsc.bitcast(rows[2, t, sl], jnp.bfloat16)
                        v3 = plsc.bitcast(rows[3, t, sl], jnp.bfloat16)
                        m = jnp.maximum(jnp.maximum(v0, v1),
                                        jnp.maximum(v2, v3))
                        me, mo = plsc.unpack(m, format=plsc.PackFormat.INTERLEAVED)
                        acc0 = acc0 + me * w2es[c]
                        acc1 = acc1 + mo * w2os[c]
                    tot = jnp.sum(acc0 + acc1) + b2
                    return jnp.where(lane_iota == tl, tot, resvec)

                resvec = lax.fori_loop(0, 16, token_body,
                                       jnp.zeros((16,), jnp.float32))
                out_v[pl.ds(blk * T + g * 16, 16)] = resvec
                return carry

            lax.fori_loop(0, T // 16, group_body, 0)

        def outer(i, carry):
            for par in range(NBUF):
                blk = i * NBUF + par
                wait(blk, par)
                compute_block(blk, par)

                @pl.when(blk + NBUF < nblk)
                def _():
                    issue(blk + NBUF, par)
            return carry

        lax.fori_loop(0, nblk // NBUF, outer, 0)
        pltpu.sync_copy(out_v, out_hbm.at[pl.ds(base_tok, tpw)])

    return sc_fn(h, gidx3, w2p)


def kernel(hidden_states, mask, word_set_idx, W1_w, W1_b, w2_w, w2_b):
    del mask
    b, s, d = hidden_states.shape
    x = hidden_states.reshape(b * s, d)
    h = _compute_h(x, W1_w, W1_b.reshape(1, d))

    f = word_set_idx.shape[-1]
    gidx3 = jnp.transpose(word_set_idx, (0, 2, 1)).astype(jnp.int32)
    w2p = jnp.concatenate([w2_w.reshape(-1), w2_b.reshape(-1),
                           jnp.zeros((15,), jnp.float32)])

    info = plsc.get_sparse_core_info()
    out = _sc_pool_dot(h, gidx3, w2p, b * s, d, f, info.num_cores)
    return out.reshape(b, s)


# speedup vs baseline: 1.2480x; 1.2480x over previous
import functools

import jax
import jax.numpy as jnp
from jax import lax
from jax.experimental import pallas as pl
from jax.experimental.pallas import tpu as pltpu
from jax.experimental.pallas import tpu_sc as plsc

BM = 2048
T = 64
NBUF = 2


def _mm_tanh_kernel(x_ref, w_ref, b_ref, o_ref):
    acc = lax.dot_general(x_ref[...], w_ref[...], (((1,), (1,)), ((), ())),
                          preferred_element_type=jnp.float32)
    h = jnp.tanh(acc + b_ref[...]).astype(jnp.bfloat16)
    d = h.shape[1]
    lo = lax.bitcast_convert_type(h[:, :d // 2], jnp.uint16)
    hi = lax.bitcast_convert_type(h[:, d // 2:], jnp.uint16)
    w = (hi.astype(jnp.uint32) << 16) | lo.astype(jnp.uint32)
    o_ref[...] = lax.bitcast_convert_type(w, jnp.int32)


def _compute_h(x, w1, b1):
    n, d = x.shape
    return pl.pallas_call(
        _mm_tanh_kernel,
        grid=(n // BM,),
        in_specs=[
            pl.BlockSpec((BM, d), lambda i: (i, 0)),
            pl.BlockSpec((d, d), lambda i: (0, 0)),
            pl.BlockSpec((1, d), lambda i: (0, 0)),
        ],
        out_specs=pl.BlockSpec((BM, d // 2), lambda i: (i, 0)),
        out_shape=jax.ShapeDtypeStruct((n, d // 2), jnp.int32),
    )(x, w1, b1)


@functools.partial(jax.jit, static_argnums=(3, 4, 5, 6))
def _sc_pool_dot(h, gidx3, w2p, ntok, d, f_, nc):
    ns = 16
    nw = nc * ns
    tpw = ntok // nw
    nblk = tpw // T
    dw = d // 2
    nch = dw // 16
    seq = gidx3.shape[-1]

    mesh = plsc.VectorSubcoreMesh(core_axis_name="c", subcore_axis_name="s")

    @functools.partial(
        pl.kernel,
        out_type=jax.ShapeDtypeStruct((ntok,), jnp.float32),
        mesh=mesh,
        scratch_types=[
            pltpu.VMEM((NBUF, f_, T, dw), jnp.int32),
            pltpu.VMEM((f_, tpw), jnp.int32),
            pltpu.VMEM((d + 16,), jnp.float32),
            pltpu.VMEM((tpw,), jnp.float32),
            pltpu.SemaphoreType.DMA,
            pltpu.SemaphoreType.DMA,
        ],
        compiler_params=pltpu.CompilerParams(use_tc_tiling_on_sc=False,
                                             needs_layout_passes=False),
    )
    def sc_fn(h_hbm, gidx_hbm, w2p_hbm, out_hbm, rows_v, idx_v, w2_v, out_v,
              sem0, sem1):
        wid = lax.axis_index("s") * nc + lax.axis_index("c")
        base_tok = wid * tpw
        bat = base_tok // seq
        s0 = base_tok - bat * seq
        sems = [sem0, sem1]

        pltpu.sync_copy(w2p_hbm, w2_v)
        b2 = w2_v[pl.ds(d, 16)][0]
        w2es = [w2_v[pl.ds(16 * c, 16)] for c in range(nch)]
        w2os = [w2_v[pl.ds(dw + 16 * c, 16)] for c in range(nch)]

        for ff in range(f_):
            pltpu.sync_copy(gidx_hbm.at[bat, ff, pl.ds(s0, tpw)],
                            idx_v.at[ff])
        boff = jnp.zeros((16,), jnp.int32) + bat * seq

        def off_body(i, carry):
            for ff in range(f_):
                sl = pl.ds(i * 16, 16)
                idx_v[ff, sl] = idx_v[ff, sl] + boff
            return carry

        lax.fori_loop(0, tpw // 16, off_body, 0)

        def issue(blk, s):
            for ff in range(f_):
                pltpu.async_copy(
                    h_hbm.at[idx_v.at[ff, pl.ds(blk * T, T)]],
                    rows_v.at[s, ff], sems[s])

        def wait(blk, s):
            for ff in range(f_):
                pltpu.make_async_copy(
                    h_hbm.at[idx_v.at[ff, pl.ds(blk * T, T)]],
                    rows_v.at[s, ff], sems[s]).wait()

        issue(0, 0)
        issue(1, 1)

        lane_iota = lax.iota(jnp.int32, 16)

        def compute_block(blk, s):
            rows = rows_v.at[s]

            def group_body(g, carry):
                def token_body(tl, resvec):
                    t = g * 16 + tl
                    acc0 = jnp.zeros((16,), jnp.float32)
                    acc1 = jnp.zeros((16,), jnp.float32)
                    for c in range(nch):
                        sl = pl.ds(16 * c, 16)
                        v0 = plsc.bitcast(rows[0, t, sl], jnp.bfloat16)
                        v1 = plsc.bitcast(rows[1, t, sl], jnp.bfloat16)
                        v2 = plsc.bitcast(rows[2, t, sl], jnp.bfloat16)
                        v3 = plsc.bitcast(rows[3, t, sl], jnp.bfloat16)
                        m = jnp.maximum(jnp.maximum(v0, v1),
                                        jnp.maximum(v2, v3))
                        me, mo = plsc.unpack(m, format=plsc.PackFormat.INTERLEAVED)
                        acc0 = acc0 + me * w2es[c]
                        acc1 = acc1 + mo * w2os[c]
                    tot = jnp.sum(acc0 + acc1) + b2
                    return jnp.where(lane_iota == tl, tot, resvec)

                resvec = lax.fori_loop(0, 16, token_body,
                                       jnp.zeros((16,), jnp.float32))
                out_v[pl.ds(blk * T + g * 16, 16)] = resvec
                return carry

            lax.fori_loop(0, T // 16, group_body, 0)

        def outer(i, carry):
            for par in range(NBUF):
                blk = i * NBUF + par
                wait(blk, par)
                compute_block(blk, par)

                @pl.when(blk + NBUF < nblk)
                def _():
                    issue(blk + NBUF, par)
            return carry

        lax.fori_loop(0, nblk // NBUF, outer, 0)
        pltpu.sync_copy(out_v, out_hbm.at[pl.ds(base_tok, tpw)])

    return sc_fn(h, gidx3, w2p)


def kernel(hidden_states, mask, word_set_idx, W1_w, W1_b, w2_w, w2_b):
    del mask
    b, s, d = hidden_states.shape
    x = hidden_states.reshape(b * s, d)
    h = _compute_h(x, W1_w, W1_b.reshape(1, d))

    f = word_set_idx.shape[-1]
    gidx3 = jnp.transpose(word_set_idx, (0, 2, 1)).astype(jnp.int32)
    w2p = jnp.concatenate([w2_w.reshape(-1), w2_b.reshape(-1),
                           jnp.zeros((15,), jnp.float32)])

    info = plsc.get_sparse_core_info()
    out = _sc_pool_dot(h, gidx3, w2p, b * s, d, f, info.num_cores)
    return out.reshape(b, s)


# speedup vs baseline: 1.2914x; 1.0347x over previous
import functools

import jax
import jax.numpy as jnp
from jax import lax
from jax.experimental import pallas as pl
from jax.experimental.pallas import tpu as pltpu
from jax.experimental.pallas import tpu_sc as plsc

BM = 4096
T = 64
NBUF = 2


def _mm_tanh_kernel(x_ref, w_ref, b_ref, o_ref):
    acc = lax.dot_general(x_ref[...], w_ref[...], (((1,), (1,)), ((), ())),
                          preferred_element_type=jnp.float32)
    h = jnp.tanh(acc + b_ref[...]).astype(jnp.bfloat16)
    d = h.shape[1]
    lo = lax.bitcast_convert_type(h[:, :d // 2], jnp.uint16)
    hi = lax.bitcast_convert_type(h[:, d // 2:], jnp.uint16)
    w = (hi.astype(jnp.uint32) << 16) | lo.astype(jnp.uint32)
    o_ref[...] = lax.bitcast_convert_type(w, jnp.int32)


def _compute_h(x, w1, b1):
    n, d = x.shape
    return pl.pallas_call(
        _mm_tanh_kernel,
        grid=(n // BM,),
        in_specs=[
            pl.BlockSpec((BM, d), lambda i: (i, 0)),
            pl.BlockSpec((d, d), lambda i: (0, 0)),
            pl.BlockSpec((1, d), lambda i: (0, 0)),
        ],
        out_specs=pl.BlockSpec((BM, d // 2), lambda i: (i, 0)),
        out_shape=jax.ShapeDtypeStruct((n, d // 2), jnp.int32),
    )(x, w1, b1)


@functools.partial(jax.jit, static_argnums=(3, 4, 5, 6))
def _sc_pool_dot(h, gidx3, w2p, ntok, d, f_, nc):
    ns = 16
    nw = nc * ns
    tpw = ntok // nw
    nblk = tpw // T
    dw = d // 2
    nch = dw // 16
    seq = gidx3.shape[-1]

    mesh = plsc.VectorSubcoreMesh(core_axis_name="c", subcore_axis_name="s")

    @functools.partial(
        pl.kernel,
        out_type=jax.ShapeDtypeStruct((ntok,), jnp.float32),
        mesh=mesh,
        scratch_types=[
            pltpu.VMEM((NBUF, f_, T, dw), jnp.int32),
            pltpu.VMEM((f_, tpw), jnp.int32),
            pltpu.VMEM((d + 16,), jnp.float32),
            pltpu.VMEM((tpw,), jnp.float32),
            pltpu.SemaphoreType.DMA,
            pltpu.SemaphoreType.DMA,
        ],
        compiler_params=pltpu.CompilerParams(use_tc_tiling_on_sc=False,
                                             needs_layout_passes=False),
    )
    def sc_fn(h_hbm, gidx_hbm, w2p_hbm, out_hbm, rows_v, idx_v, w2_v, out_v,
              sem0, sem1):
        wid = lax.axis_index("s") * nc + lax.axis_index("c")
        base_tok = wid * tpw
        bat = base_tok // seq
        s0 = base_tok - bat * seq
        sems = [sem0, sem1]

        pltpu.sync_copy(w2p_hbm, w2_v)
        b2 = w2_v[pl.ds(d, 16)][0]
        w2es = [w2_v[pl.ds(16 * c, 16)] for c in range(nch)]
        w2os = [w2_v[pl.ds(dw + 16 * c, 16)] for c in range(nch)]

        for ff in range(f_):
            pltpu.sync_copy(gidx_hbm.at[bat, ff, pl.ds(s0, tpw)],
                            idx_v.at[ff])
        boff = jnp.zeros((16,), jnp.int32) + bat * seq

        def off_body(i, carry):
            for ff in range(f_):
                sl = pl.ds(i * 16, 16)
                idx_v[ff, sl] = idx_v[ff, sl] + boff
            return carry

        lax.fori_loop(0, tpw // 16, off_body, 0)

        def issue(blk, s):
            for ff in range(f_):
                pltpu.async_copy(
                    h_hbm.at[idx_v.at[ff, pl.ds(blk * T, T)]],
                    rows_v.at[s, ff], sems[s])

        def wait(blk, s):
            for ff in range(f_):
                pltpu.make_async_copy(
                    h_hbm.at[idx_v.at[ff, pl.ds(blk * T, T)]],
                    rows_v.at[s, ff], sems[s]).wait()

        issue(0, 0)
        issue(1, 1)

        lane_iota = lax.iota(jnp.int32, 16)

        def compute_block(blk, s):
            rows = rows_v.at[s]

            def group_body(g, carry):
                def token_body(tl, resvec):
                    t = g * 16 + tl
                    acc0 = jnp.zeros((16,), jnp.float32)
                    acc1 = jnp.zeros((16,), jnp.float32)
                    for c in range(nch):
                        sl = pl.ds(16 * c, 16)
                        v0 = plsc.bitcast(rows[0, t, sl], jnp.bfloat16)
                        v1 = plsc.bitcast(rows[1, t, sl], jnp.bfloat16)
                        v2 = plsc.bitcast(rows[2, t, sl], jnp.bfloat16)
                        v3 = plsc.bitcast(rows[3, t, sl], jnp.bfloat16)
                        m = jnp.maximum(jnp.maximum(v0, v1),
                                        jnp.maximum(v2, v3))
                        me, mo = plsc.unpack(m, format=plsc.PackFormat.INTERLEAVED)
                        acc0 = acc0 + me * w2es[c]
                        acc1 = acc1 + mo * w2os[c]
                    tot = jnp.sum(acc0 + acc1) + b2
                    return jnp.where(lane_iota == tl, tot, resvec)

                resvec = lax.fori_loop(0, 16, token_body,
                                       jnp.zeros((16,), jnp.float32))
                out_v[pl.ds(blk * T + g * 16, 16)] = resvec
                return carry

            lax.fori_loop(0, T // 16, group_body, 0)

        def outer(i, carry):
            for par in range(NBUF):
                blk = i * NBUF + par
                wait(blk, par)
                compute_block(blk, par)

                @pl.when(blk + NBUF < nblk)
                def _():
                    issue(blk + NBUF, par)
            return carry

        lax.fori_loop(0, nblk // NBUF, outer, 0)
        pltpu.sync_copy(out_v, out_hbm.at[pl.ds(base_tok, tpw)])

    return sc_fn(h, gidx3, w2p)


def kernel(hidden_states, mask, word_set_idx, W1_w, W1_b, w2_w, w2_b):
    del mask
    b, s, d = hidden_states.shape
    x = hidden_states.reshape(b * s, d)
    h = _compute_h(x, W1_w, W1_b.reshape(1, d))

    f = word_set_idx.shape[-1]
    gidx3 = jnp.transpose(word_set_idx, (0, 2, 1)).astype(jnp.int32)
    w2p = jnp.concatenate([w2_w.reshape(-1), w2_b.reshape(-1),
                           jnp.zeros((15,), jnp.float32)])

    info = plsc.get_sparse_core_info()
    out = _sc_pool_dot(h, gidx3, w2p, b * s, d, f, info.num_cores)
    return out.reshape(b, s)


# speedup vs baseline: 1.3215x; 1.0233x over previous
import functools

import jax
import jax.numpy as jnp
from jax import lax
from jax.experimental import pallas as pl
from jax.experimental.pallas import tpu as pltpu
from jax.experimental.pallas import tpu_sc as plsc

BM = 8192
T = 64
NBUF = 2


def _mm_tanh_kernel(x_ref, w_ref, b_ref, o_ref):
    acc = lax.dot_general(x_ref[...], w_ref[...], (((1,), (1,)), ((), ())),
                          preferred_element_type=jnp.float32)
    h = jnp.tanh(acc + b_ref[...]).astype(jnp.bfloat16)
    d = h.shape[1]
    lo = lax.bitcast_convert_type(h[:, :d // 2], jnp.uint16)
    hi = lax.bitcast_convert_type(h[:, d // 2:], jnp.uint16)
    w = (hi.astype(jnp.uint32) << 16) | lo.astype(jnp.uint32)
    o_ref[...] = lax.bitcast_convert_type(w, jnp.int32)


def _compute_h(x, w1, b1):
    n, d = x.shape
    return pl.pallas_call(
        _mm_tanh_kernel,
        grid=(n // BM,),
        in_specs=[
            pl.BlockSpec((BM, d), lambda i: (i, 0)),
            pl.BlockSpec((d, d), lambda i: (0, 0)),
            pl.BlockSpec((1, d), lambda i: (0, 0)),
        ],
        out_specs=pl.BlockSpec((BM, d // 2), lambda i: (i, 0)),
        out_shape=jax.ShapeDtypeStruct((n, d // 2), jnp.int32),
    )(x, w1, b1)


@functools.partial(jax.jit, static_argnums=(3, 4, 5, 6))
def _sc_pool_dot(h, gidx3, w2p, ntok, d, f_, nc):
    ns = 16
    nw = nc * ns
    tpw = ntok // nw
    nblk = tpw // T
    dw = d // 2
    nch = dw // 16
    seq = gidx3.shape[-1]

    mesh = plsc.VectorSubcoreMesh(core_axis_name="c", subcore_axis_name="s")

    @functools.partial(
        pl.kernel,
        out_type=jax.ShapeDtypeStruct((ntok,), jnp.float32),
        mesh=mesh,
        scratch_types=[
            pltpu.VMEM((NBUF, f_, T, dw), jnp.int32),
            pltpu.VMEM((f_, tpw), jnp.int32),
            pltpu.VMEM((d + 16,), jnp.float32),
            pltpu.VMEM((tpw,), jnp.float32),
            pltpu.SemaphoreType.DMA,
            pltpu.SemaphoreType.DMA,
        ],
        compiler_params=pltpu.CompilerParams(use_tc_tiling_on_sc=False,
                                             needs_layout_passes=False),
    )
    def sc_fn(h_hbm, gidx_hbm, w2p_hbm, out_hbm, rows_v, idx_v, w2_v, out_v,
              sem0, sem1):
        wid = lax.axis_index("s") * nc + lax.axis_index("c")
        base_tok = wid * tpw
        bat = base_tok // seq
        s0 = base_tok - bat * seq
        sems = [sem0, sem1]

        pltpu.sync_copy(w2p_hbm, w2_v)
        b2 = w2_v[pl.ds(d, 16)][0]
        w2es = [w2_v[pl.ds(16 * c, 16)] for c in range(nch)]
        w2os = [w2_v[pl.ds(dw + 16 * c, 16)] for c in range(nch)]

        for ff in range(f_):
            pltpu.sync_copy(gidx_hbm.at[bat, ff, pl.ds(s0, tpw)],
                            idx_v.at[ff])
        boff = jnp.zeros((16,), jnp.int32) + bat * seq

        def off_body(i, carry):
            for ff in range(f_):
                sl = pl.ds(i * 16, 16)
                idx_v[ff, sl] = idx_v[ff, sl] + boff
            return carry

        lax.fori_loop(0, tpw // 16, off_body, 0)

        def issue(blk, s):
            for ff in range(f_):
                pltpu.async_copy(
                    h_hbm.at[idx_v.at[ff, pl.ds(blk * T, T)]],
                    rows_v.at[s, ff], sems[s])

        def wait(blk, s):
            for ff in range(f_):
                pltpu.make_async_copy(
                    h_hbm.at[idx_v.at[ff, pl.ds(blk * T, T)]],
                    rows_v.at[s, ff], sems[s]).wait()

        issue(0, 0)
        issue(1, 1)

        lane_iota = lax.iota(jnp.int32, 16)

        def compute_block(blk, s):
            rows = rows_v.at[s]

            def group_body(g, carry):
                def token_body(tl, resvec):
                    t = g * 16 + tl
                    acc0 = jnp.zeros((16,), jnp.float32)
                    acc1 = jnp.zeros((16,), jnp.float32)
                    for c in range(nch):
                        sl = pl.ds(16 * c, 16)
                        v0 = plsc.bitcast(rows[0, t, sl], jnp.bfloat16)
                        v1 = plsc.bitcast(rows[1, t, sl], jnp.bfloat16)
                        v2 = plsc.bitcast(rows[2, t, sl], jnp.bfloat16)
                        v3 = plsc.bitcast(rows[3, t, sl], jnp.bfloat16)
                        m = jnp.maximum(jnp.maximum(v0, v1),
                                        jnp.maximum(v2, v3))
                        me, mo = plsc.unpack(m, format=plsc.PackFormat.INTERLEAVED)
                        acc0 = acc0 + me * w2es[c]
                        acc1 = acc1 + mo * w2os[c]
                    tot = jnp.sum(acc0 + acc1) + b2
                    return jnp.where(lane_iota == tl, tot, resvec)

                resvec = lax.fori_loop(0, 16, token_body,
                                       jnp.zeros((16,), jnp.float32))
                out_v[pl.ds(blk * T + g * 16, 16)] = resvec
                return carry

            lax.fori_loop(0, T // 16, group_body, 0)

        def outer(i, carry):
            for par in range(NBUF):
                blk = i * NBUF + par
                wait(blk, par)
                compute_block(blk, par)

                @pl.when(blk + NBUF < nblk)
                def _():
                    issue(blk + NBUF, par)
            return carry

        lax.fori_loop(0, nblk // NBUF, outer, 0)
        pltpu.sync_copy(out_v, out_hbm.at[pl.ds(base_tok, tpw)])

    return sc_fn(h, gidx3, w2p)


def kernel(hidden_states, mask, word_set_idx, W1_w, W1_b, w2_w, w2_b):
    del mask
    b, s, d = hidden_states.shape
    x = hidden_states.reshape(b * s, d)
    h = _compute_h(x, W1_w, W1_b.reshape(1, d))

    f = word_set_idx.shape[-1]
    gidx3 = jnp.transpose(word_set_idx, (0, 2, 1)).astype(jnp.int32)
    w2p = jnp.concatenate([w2_w.reshape(-1), w2_b.reshape(-1),
                           jnp.zeros((15,), jnp.float32)])

    info = plsc.get_sparse_core_info()
    out = _sc_pool_dot(h, gidx3, w2p, b * s, d, f, info.num_cores)
    return out.reshape(b, s)
